# baseline (speedup 1.0000x reference)
import jax
import jax.numpy as jnp
from jax.experimental import pallas as pl
from jax.experimental.pallas import tpu as pltpu

_SEQ = 512
_PRED = 96
_D = 512
_LAYERS = 2
_NV = 321
_B = 32
_G = 2
_K = 8
_DFF = 4 * _D
_NP = 384
_PP = 128

_HI = jax.lax.Precision.HIGHEST


def _dot(a, b):
    return jax.lax.dot_general(
        a, b, (((1,), (0,)), ((), ())), preferred_element_type=jnp.float32)


def _dot_nt(a, b):
    return jax.lax.dot_general(
        a, b, (((1,), (1,)), ((), ())), preferred_element_type=jnp.float32)


def _mm_nt_hi(a, b):
    return jax.lax.dot_general(
        a, b, (((1,), (1,)), ((), ())),
        preferred_element_type=jnp.float32, precision=_HI)


def _mm_tn_hi(a, b):
    return jax.lax.dot_general(
        a, b, (((0,), (0,)), ((), ())),
        preferred_element_type=jnp.float32, precision=_HI)


def _split(a):
    ah = a.astype(jnp.bfloat16)
    al = (a - ah.astype(jnp.float32)).astype(jnp.bfloat16)
    return ah, al


def _mm3w(a, wh, wl):
    ah, al = _split(a)
    return _dot(ah, wh) + (_dot(al, wh) + _dot(ah, wl))


def _mm3_nt(a, b):
    ah, al = _split(a)
    bh, bl = _split(b)
    return _dot_nt(ah, bh) + (_dot_nt(al, bh) + _dot_nt(ah, bl))


def _mm3(a, b):
    ah, al = _split(a)
    bh, bl = _split(b)
    return _dot(ah, bh) + (_dot(al, bh) + _dot(ah, bl))


def _layer_norm(x, s, b):
    mu = jnp.mean(x, axis=-1, keepdims=True)
    var = jnp.mean((x - mu) ** 2, axis=-1, keepdims=True)
    return (x - mu) / jnp.sqrt(var + 1e-5) * s + b


def _cluster_onehot(x):
    rows = jax.lax.broadcasted_iota(jnp.int32, (_NP, 1), 0)
    valid = rows < _NV
    kcol = jax.lax.broadcasted_iota(jnp.int32, (_NP, _K), 1)
    xm = x - jnp.mean(x, axis=1, keepdims=True)
    nrm = jnp.sqrt(jnp.sum(xm * xm, axis=1, keepdims=True))
    xn = jnp.where(valid, xm / (nrm + 1e-8), 0.0)
    a2 = jnp.sum(xn * xn, axis=1, keepdims=True)
    cent = xn[:_K]
    oh = None
    for it in range(6):
        ab = _mm_nt_hi(xn, cent)
        c2 = jnp.sum(cent * cent, axis=1, keepdims=True)
        dist = (a2 - 2.0 * ab) + jnp.transpose(c2)
        best = jnp.min(dist, axis=1, keepdims=True)
        besti = jnp.min(jnp.where(dist == best, kcol, _K),
                        axis=1, keepdims=True)
        oh = jnp.where((besti == kcol) & valid, 1.0, 0.0)
        if it == 5:
            break
        sums = _mm_tn_hi(oh, xn)
        counts = _mm_tn_hi(oh, jnp.ones((_NP, 1), jnp.float32))
        cent = sums / (counts + 1e-8)
    return oh


def _fused_kernel(x_ref, wembh_ref, wembl_ref, bemb_ref,
                  wqh_ref, wql_ref, bq_ref, wkh_ref, wkl_ref, bk_ref,
                  wvh_ref, wvl_ref, bv_ref, woh_ref, wol_ref, bo_ref,
                  l1s_ref, l1b_ref, w1h_ref, w1l_ref, b1_ref,
                  w2h_ref, w2l_ref, b2_ref, l2s_ref, l2b_ref,
                  lfs_ref, lfb_ref, wdech_ref, wdecl_ref, bdec_ref,
                  out_ref):
    masks = []
    xs_list = []
    mean_list = []
    std_list = []
    for g in range(_G):
        x = x_ref[g]
        oh = _cluster_onehot(x)
        ohb = oh.astype(jnp.bfloat16)
        maskf = _dot_nt(ohb, ohb)
        masks.append(maskf > 0.5)

        means = jnp.mean(x, axis=1, keepdims=True)
        xc = x - means
        var = jnp.mean(xc * xc, axis=1, keepdims=True)
        stdev = jnp.sqrt(var + 1e-5)
        xs_list.append(xc / stdev)
        mean_list.append(means)
        std_list.append(stdev)

    xs = jnp.concatenate(xs_list, axis=0)
    enc = _mm3w(xs, wembh_ref[...], wembl_ref[...]) + bemb_ref[...]
    scale = 1.0 / jnp.sqrt(jnp.float32(_D))
    for l in range(_LAYERS):
        q = _mm3w(enc, wqh_ref[l], wql_ref[l]) + bq_ref[l]
        k = _mm3w(enc, wkh_ref[l], wkl_ref[l]) + bk_ref[l]
        v = _mm3w(enc, wvh_ref[l], wvl_ref[l]) + bv_ref[l]
        avs = []
        for g in range(_G):
            sl = slice(g * _NP, (g + 1) * _NP)
            s = _mm3_nt(q[sl], k[sl]) * scale
            s = jnp.where(masks[g], s, jnp.float32(-1e9))
            m = jnp.max(s, axis=1, keepdims=True)
            e = jnp.exp(s - m)
            attn = e / jnp.sum(e, axis=1, keepdims=True)
            avs.append(_mm3(attn, v[sl]))
        av = jnp.concatenate(avs, axis=0)
        o = _mm3w(av, woh_ref[l], wol_ref[l]) + bo_ref[l]
        enc = _layer_norm(enc + o, l1s_ref[l], l1b_ref[l])
        h = jnp.maximum(_mm3w(enc, w1h_ref[l], w1l_ref[l]) + b1_ref[l], 0.0)
        h = _mm3w(h, w2h_ref[l], w2l_ref[l]) + b2_ref[l]
        enc = _layer_norm(enc + h, l2s_ref[l], l2b_ref[l])
    enc = _layer_norm(enc, lfs_ref[...], lfb_ref[...])
    dec = _mm3w(enc, wdech_ref[...], wdecl_ref[...]) + bdec_ref[...]
    stdev_all = jnp.concatenate(std_list, axis=0)
    means_all = jnp.concatenate(mean_list, axis=0)
    dec = dec * stdev_all + means_all
    for g in range(_G):
        out_ref[g] = dec[g * _NP:(g + 1) * _NP]


def _full(shape):
    nd = len(shape)
    return pl.BlockSpec(shape, lambda b: (0,) * nd)


@jax.jit
def kernel(input_arr, params):
    x_t = jnp.transpose(input_arr, (0, 2, 1))
    x_p = jnp.pad(x_t, ((0, 0), (0, _NP - _NV), (0, 0)))

    p = params
    bemb = p['b_emb'].reshape(1, _D)
    bq = p['bq'].reshape(_LAYERS, 1, _D)
    bk = p['bk'].reshape(_LAYERS, 1, _D)
    bv = p['bv'].reshape(_LAYERS, 1, _D)
    bo = p['bo'].reshape(_LAYERS, 1, _D)
    l1s = p['ln1_s'].reshape(_LAYERS, 1, _D)
    l1b = p['ln1_b'].reshape(_LAYERS, 1, _D)
    b1 = p['b1'].reshape(_LAYERS, 1, _DFF)
    b2 = p['b2'].reshape(_LAYERS, 1, _D)
    l2s = p['ln2_s'].reshape(_LAYERS, 1, _D)
    l2b = p['ln2_b'].reshape(_LAYERS, 1, _D)
    lfs = p['lnf_s'].reshape(1, _D)
    lfb = p['lnf_b'].reshape(1, _D)
    wdec = jnp.pad(p['Wdec'], ((0, 0), (0, _PP - _PRED)))
    bdec = jnp.pad(p['bdec'], (0, _PP - _PRED)).reshape(1, _PP)

    def split(w):
        wh = w.astype(jnp.bfloat16)
        wl = (w - wh.astype(jnp.float32)).astype(jnp.bfloat16)
        return wh, wl

    wembh, wembl = split(p['W_emb'])
    wqh, wql = split(p['Wq'])
    wkh, wkl = split(p['Wk'])
    wvh, wvl = split(p['Wv'])
    woh, wol = split(p['Wo'])
    w1h, w1l = split(p['W1'])
    w2h, w2l = split(p['W2'])
    wdech, wdecl = split(wdec)

    operands = [
        x_p, wembh, wembl, bemb,
        wqh, wql, bq, wkh, wkl, bk, wvh, wvl, bv, woh, wol, bo,
        l1s, l1b, w1h, w1l, b1, w2h, w2l, b2, l2s, l2b,
        lfs, lfb, wdech, wdecl, bdec,
    ]
    in_specs = [pl.BlockSpec((_G, _NP, _SEQ), lambda b: (b, 0, 0))]
    in_specs += [_full(op.shape) for op in operands[1:]]

    out = pl.pallas_call(
        _fused_kernel,
        grid=(_B // _G,),
        in_specs=in_specs,
        out_specs=pl.BlockSpec((_G, _NP, _PP), lambda b: (b, 0, 0)),
        out_shape=jax.ShapeDtypeStruct((_B, _NP, _PP), jnp.float32),
        compiler_params=pltpu.CompilerParams(
            dimension_semantics=('parallel',)),
    )(*operands)
    return jnp.transpose(out[:, :_NV, :_PRED], (0, 2, 1))


# speedup vs baseline: 1.1694x; 1.1694x over previous
import numpy as np

import jax
import jax.numpy as jnp
from jax.experimental import pallas as pl
from jax.experimental.pallas import tpu as pltpu
from jax.sharding import Mesh, PartitionSpec as P

_SEQ = 512
_PRED = 96
_D = 512
_LAYERS = 2
_NV = 321
_B = 32
_G = 2
_K = 8
_DFF = 4 * _D
_NP = 384
_PP = 128

_HI = jax.lax.Precision.HIGHEST


def _dot(a, b):
    return jax.lax.dot_general(
        a, b, (((1,), (0,)), ((), ())), preferred_element_type=jnp.float32)


def _dot_nt(a, b):
    return jax.lax.dot_general(
        a, b, (((1,), (1,)), ((), ())), preferred_element_type=jnp.float32)


def _mm_nt_hi(a, b):
    return jax.lax.dot_general(
        a, b, (((1,), (1,)), ((), ())),
        preferred_element_type=jnp.float32, precision=_HI)


def _mm_tn_hi(a, b):
    return jax.lax.dot_general(
        a, b, (((0,), (0,)), ((), ())),
        preferred_element_type=jnp.float32, precision=_HI)


def _split(a):
    ah = a.astype(jnp.bfloat16)
    al = (a - ah.astype(jnp.float32)).astype(jnp.bfloat16)
    return ah, al


def _mm3w(a, wh, wl):
    ah, al = _split(a)
    return _dot(ah, wh) + (_dot(al, wh) + _dot(ah, wl))


def _mm3_nt(a, b):
    ah, al = _split(a)
    bh, bl = _split(b)
    return _dot_nt(ah, bh) + (_dot_nt(al, bh) + _dot_nt(ah, bl))


def _mm3(a, b):
    ah, al = _split(a)
    bh, bl = _split(b)
    return _dot(ah, bh) + (_dot(al, bh) + _dot(ah, bl))


def _layer_norm(x, s, b):
    mu = jnp.mean(x, axis=-1, keepdims=True)
    var = jnp.mean((x - mu) ** 2, axis=-1, keepdims=True)
    return (x - mu) / jnp.sqrt(var + 1e-5) * s + b


def _cluster_onehot(x):
    rows = jax.lax.broadcasted_iota(jnp.int32, (_NP, 1), 0)
    valid = rows < _NV
    kcol = jax.lax.broadcasted_iota(jnp.int32, (_NP, _K), 1)
    xm = x - jnp.mean(x, axis=1, keepdims=True)
    nrm = jnp.sqrt(jnp.sum(xm * xm, axis=1, keepdims=True))
    xn = jnp.where(valid, xm / (nrm + 1e-8), 0.0)
    a2 = jnp.sum(xn * xn, axis=1, keepdims=True)
    cent = xn[:_K]
    oh = None
    for it in range(6):
        ab = _mm_nt_hi(xn, cent)
        c2 = jnp.sum(cent * cent, axis=1, keepdims=True)
        dist = (a2 - 2.0 * ab) + jnp.transpose(c2)
        best = jnp.min(dist, axis=1, keepdims=True)
        besti = jnp.min(jnp.where(dist == best, kcol, _K),
                        axis=1, keepdims=True)
        oh = jnp.where((besti == kcol) & valid, 1.0, 0.0)
        if it == 5:
            break
        sums = _mm_tn_hi(oh, xn)
        counts = _mm_tn_hi(oh, jnp.ones((_NP, 1), jnp.float32))
        cent = sums / (counts + 1e-8)
    return oh


def _fused_kernel(x_ref, wembh_ref, wembl_ref, bemb_ref,
                  wqh_ref, wql_ref, bq_ref, wkh_ref, wkl_ref, bk_ref,
                  wvh_ref, wvl_ref, bv_ref, woh_ref, wol_ref, bo_ref,
                  l1s_ref, l1b_ref, w1h_ref, w1l_ref, b1_ref,
                  w2h_ref, w2l_ref, b2_ref, l2s_ref, l2b_ref,
                  lfs_ref, lfb_ref, wdech_ref, wdecl_ref, bdec_ref,
                  out_ref):
    masks = []
    xs_list = []
    mean_list = []
    std_list = []
    for g in range(_G):
        x = x_ref[g]
        oh = _cluster_onehot(x)
        ohb = oh.astype(jnp.bfloat16)
        maskf = _dot_nt(ohb, ohb)
        masks.append(maskf > 0.5)

        means = jnp.mean(x, axis=1, keepdims=True)
        xc = x - means
        var = jnp.mean(xc * xc, axis=1, keepdims=True)
        stdev = jnp.sqrt(var + 1e-5)
        xs_list.append(xc / stdev)
        mean_list.append(means)
        std_list.append(stdev)

    xs = jnp.concatenate(xs_list, axis=0)
    enc = _mm3w(xs, wembh_ref[...], wembl_ref[...]) + bemb_ref[...]
    scale = 1.0 / jnp.sqrt(jnp.float32(_D))
    for l in range(_LAYERS):
        q = _mm3w(enc, wqh_ref[l], wql_ref[l]) + bq_ref[l]
        k = _mm3w(enc, wkh_ref[l], wkl_ref[l]) + bk_ref[l]
        v = _mm3w(enc, wvh_ref[l], wvl_ref[l]) + bv_ref[l]
        avs = []
        for g in range(_G):
            sl = slice(g * _NP, (g + 1) * _NP)
            s = _mm3_nt(q[sl], k[sl]) * scale
            s = jnp.where(masks[g], s, jnp.float32(-1e9))
            m = jnp.max(s, axis=1, keepdims=True)
            e = jnp.exp(s - m)
            attn = e / jnp.sum(e, axis=1, keepdims=True)
            avs.append(_mm3(attn, v[sl]))
        av = jnp.concatenate(avs, axis=0)
        o = _mm3w(av, woh_ref[l], wol_ref[l]) + bo_ref[l]
        enc = _layer_norm(enc + o, l1s_ref[l], l1b_ref[l])
        h = jnp.maximum(_mm3w(enc, w1h_ref[l], w1l_ref[l]) + b1_ref[l], 0.0)
        h = _mm3w(h, w2h_ref[l], w2l_ref[l]) + b2_ref[l]
        enc = _layer_norm(enc + h, l2s_ref[l], l2b_ref[l])
    enc = _layer_norm(enc, lfs_ref[...], lfb_ref[...])
    dec = _mm3w(enc, wdech_ref[...], wdecl_ref[...]) + bdec_ref[...]
    stdev_all = jnp.concatenate(std_list, axis=0)
    means_all = jnp.concatenate(mean_list, axis=0)
    dec = dec * stdev_all + means_all
    for g in range(_G):
        out_ref[g] = dec[g * _NP:(g + 1) * _NP]


def _full(shape):
    nd = len(shape)
    return pl.BlockSpec(shape, lambda b: (0,) * nd)


def _run(input_arr, params):
    x_t = jnp.transpose(input_arr, (0, 2, 1))
    x_p = jnp.pad(x_t, ((0, 0), (0, _NP - _NV), (0, 0)))

    p = params
    bemb = p['b_emb'].reshape(1, _D)
    bq = p['bq'].reshape(_LAYERS, 1, _D)
    bk = p['bk'].reshape(_LAYERS, 1, _D)
    bv = p['bv'].reshape(_LAYERS, 1, _D)
    bo = p['bo'].reshape(_LAYERS, 1, _D)
    l1s = p['ln1_s'].reshape(_LAYERS, 1, _D)
    l1b = p['ln1_b'].reshape(_LAYERS, 1, _D)
    b1 = p['b1'].reshape(_LAYERS, 1, _DFF)
    b2 = p['b2'].reshape(_LAYERS, 1, _D)
    l2s = p['ln2_s'].reshape(_LAYERS, 1, _D)
    l2b = p['ln2_b'].reshape(_LAYERS, 1, _D)
    lfs = p['lnf_s'].reshape(1, _D)
    lfb = p['lnf_b'].reshape(1, _D)
    wdec = jnp.pad(p['Wdec'], ((0, 0), (0, _PP - _PRED)))
    bdec = jnp.pad(p['bdec'], (0, _PP - _PRED)).reshape(1, _PP)

    def split(w):
        wh = w.astype(jnp.bfloat16)
        wl = (w - wh.astype(jnp.float32)).astype(jnp.bfloat16)
        return wh, wl

    wembh, wembl = split(p['W_emb'])
    wqh, wql = split(p['Wq'])
    wkh, wkl = split(p['Wk'])
    wvh, wvl = split(p['Wv'])
    woh, wol = split(p['Wo'])
    w1h, w1l = split(p['W1'])
    w2h, w2l = split(p['W2'])
    wdech, wdecl = split(wdec)

    operands = [
        x_p, wembh, wembl, bemb,
        wqh, wql, bq, wkh, wkl, bk, wvh, wvl, bv, woh, wol, bo,
        l1s, l1b, w1h, w1l, b1, w2h, w2l, b2, l2s, l2b,
        lfs, lfb, wdech, wdecl, bdec,
    ]
    in_specs = [pl.BlockSpec((_G, _NP, _SEQ), lambda b: (b, 0, 0))]
    in_specs += [_full(op.shape) for op in operands[1:]]

    b_shard = x_p.shape[0]
    out = pl.pallas_call(
        _fused_kernel,
        grid=(b_shard // _G,),
        in_specs=in_specs,
        out_specs=pl.BlockSpec((_G, _NP, _PP), lambda b: (b, 0, 0)),
        out_shape=jax.ShapeDtypeStruct((b_shard, _NP, _PP), jnp.float32),
        compiler_params=pltpu.CompilerParams(
            dimension_semantics=('parallel',)),
    )(*operands)
    return jnp.transpose(out[:, :_NV, :_PRED], (0, 2, 1))


@jax.jit
def kernel(input_arr, params):
    devs = jax.devices()
    n_dev = 2 if len(devs) >= 2 else 1
    if n_dev == 1:
        return _run(input_arr, params)
    mesh = Mesh(np.array(devs[:n_dev]), ('d',))
    f = jax.shard_map(
        _run, mesh=mesh,
        in_specs=(P('d'), P()),
        out_specs=P('d'),
        check_vma=False,
    )
    return f(input_arr, params)
